# Initial kernel scaffold; baseline (speedup 1.0000x reference)
import functools

import jax
import jax.numpy as jnp
from jax import lax
from jax.experimental import pallas as pl
from jax.experimental.pallas import tpu as pltpu
from jax.experimental.pallas import tpu_sc as plsc

N = 10000
E = 320000
G = 64
H = 128

_NC = 2
_NS = 16
_NW = _NC * _NS
_PER_W = E // _NW
_CH = 80
_NCHUNK = _PER_W // _CH
_ROWS_PER_TILE = N // _NS


def _node_proj_body(nf, w1, b1, wl, bl, wr, br, nn_out, cat_out):
    x = nf[...]
    nn_out[...] = jax.nn.relu(
        jnp.dot(x, w1[...], preferred_element_type=jnp.float32) + b1[...])
    cat_out[:, :H] = jnp.dot(x, wl[...], preferred_element_type=jnp.float32) + bl[...]
    cat_out[:, H:] = jnp.dot(x, wr[...], preferred_element_type=jnp.float32) + br[...]


def _node_proj(nf, w1, b1, wl, bl, wr, br):
    bn = 2000
    nin = nf.shape[1]
    full = lambda a: pl.BlockSpec(a.shape, lambda i: (0,) * a.ndim)
    return pl.pallas_call(
        _node_proj_body,
        grid=(N // bn,),
        in_specs=[pl.BlockSpec((bn, nin), lambda i: (i, 0)),
                  full(w1), full(b1), full(wl), full(bl), full(wr), full(br)],
        out_specs=[pl.BlockSpec((bn, H), lambda i: (i, 0)),
                   pl.BlockSpec((bn, 2 * H), lambda i: (i, 0))],
        out_shape=[jax.ShapeDtypeStruct((N, H), jnp.float32),
                   jax.ShapeDtypeStruct((N, 2 * H), jnp.float32)],
    )(nf, w1, b1, wl, bl, wr, br)


def _gather_body(table, srcr, dstr, gs, gd, idx_s, idx_d, buf_s, buf_d,
                 sem_s, sem_d):
    wid = lax.axis_index("c") * _NS + lax.axis_index("s")
    base = wid * _PER_W

    def chunk(i, carry):
        off = base + i * _CH
        pltpu.sync_copy(srcr.at[pl.ds(off, _CH)], idx_s)
        pltpu.sync_copy(dstr.at[pl.ds(off, _CH)], idx_d)
        a = pltpu.async_copy(table.at[idx_s], buf_s, sem_s)
        b = pltpu.async_copy(table.at[idx_d], buf_d, sem_d)
        a.wait()
        b.wait()
        pltpu.sync_copy(buf_s, gs.at[pl.ds(off, _CH)])
        pltpu.sync_copy(buf_d, gd.at[pl.ds(off, _CH)])
        return carry

    lax.fori_loop(0, _NCHUNK, chunk, 0)


def _gather_sc(table, src, dst):
    mesh = plsc.VectorSubcoreMesh(core_axis_name="c", subcore_axis_name="s")
    k = functools.partial(
        pl.kernel,
        mesh=mesh,
        out_type=[jax.ShapeDtypeStruct((E, 2 * H), jnp.float32),
                  jax.ShapeDtypeStruct((E, 2 * H), jnp.float32)],
        scratch_types=[
            pltpu.VMEM((_CH,), jnp.int32),
            pltpu.VMEM((_CH,), jnp.int32),
            pltpu.VMEM((_CH, 2 * H), jnp.float32),
            pltpu.VMEM((_CH, 2 * H), jnp.float32),
            pltpu.SemaphoreType.DMA,
            pltpu.SemaphoreType.DMA,
        ],
    )(_gather_body)
    return k(table, src, dst)


def _scatter_body(e2n, dstr, zrows, out0, out1, idx, buf, agg, sem):
    cid = lax.axis_index("c")
    sid = lax.axis_index("s")
    pltpu.sync_copy(zrows, agg.at[pl.ds(sid * _ROWS_PER_TILE, _ROWS_PER_TILE)])
    plsc.subcore_barrier()

    base = (cid * _NS + sid) * _PER_W

    def chunk(i, carry):
        off = base + i * _CH
        pltpu.sync_copy(e2n.at[pl.ds(off, _CH)], buf)
        pltpu.sync_copy(dstr.at[pl.ds(off, _CH)], idx)
        pltpu.sync_copy(buf, agg.at[idx], add=True)
        return carry

    lax.fori_loop(0, _NCHUNK, chunk, 0)
    plsc.subcore_barrier()

    sl = pl.ds(sid * _ROWS_PER_TILE, _ROWS_PER_TILE)

    @pl.when(cid == 0)
    def _():
        pltpu.sync_copy(agg.at[sl], out0.at[sl])

    @pl.when(cid == 1)
    def _():
        pltpu.sync_copy(agg.at[sl], out1.at[sl])


def _scatter_sc(e2n, dst):
    mesh = plsc.VectorSubcoreMesh(core_axis_name="c", subcore_axis_name="s")
    zrows = jnp.zeros((_ROWS_PER_TILE, H), jnp.float32)
    k = functools.partial(
        pl.kernel,
        mesh=mesh,
        out_type=[jax.ShapeDtypeStruct((N, H), jnp.float32),
                  jax.ShapeDtypeStruct((N, H), jnp.float32)],
        scratch_types=[
            pltpu.VMEM((_CH,), jnp.int32),
            pltpu.VMEM((_CH, H), jnp.float32),
            pltpu.VMEM_SHARED((N, H), jnp.float32),
            pltpu.SemaphoreType.DMA,
        ],
    )(_scatter_body)
    return k(e2n, dst, zrows)


def _edge_body(ef, gs, gd, we2e, be2e, wu, bu, we2n, be2n, nef_out, e2n_out):
    x = ef[...]
    gsv = gs[...]
    gdv = gd[...]
    first = jax.nn.relu(gsv[:, :H] + gdv[:, H:])
    second = jax.nn.relu(gdv[:, :H] + gsv[:, H:])
    third = jax.nn.relu(
        jnp.dot(x, we2e[...], preferred_element_type=jnp.float32) + be2e[...])
    acc = jnp.dot(first, wu[:H, :], preferred_element_type=jnp.float32)
    acc += jnp.dot(second, wu[H:2 * H, :], preferred_element_type=jnp.float32)
    acc += jnp.dot(third, wu[2 * H:, :], preferred_element_type=jnp.float32)
    nef_out[...] = jax.nn.relu(acc + bu[...])
    e2n_out[...] = jax.nn.relu(
        jnp.dot(x, we2n[...], preferred_element_type=jnp.float32) + be2n[...])


def _edge_mm(ef, gs, gd, we2e, be2e, wu, bu, we2n, be2n):
    be = 2000
    ein = ef.shape[1]
    full = lambda a: pl.BlockSpec(a.shape, lambda i: (0,) * a.ndim)
    return pl.pallas_call(
        _edge_body,
        grid=(E // be,),
        in_specs=[pl.BlockSpec((be, ein), lambda i: (i, 0)),
                  pl.BlockSpec((be, 2 * H), lambda i: (i, 0)),
                  pl.BlockSpec((be, 2 * H), lambda i: (i, 0)),
                  full(we2e), full(be2e), full(wu), full(bu),
                  full(we2n), full(be2n)],
        out_specs=[pl.BlockSpec((be, H), lambda i: (i, 0)),
                   pl.BlockSpec((be, H), lambda i: (i, 0))],
        out_shape=[jax.ShapeDtypeStruct((E, H), jnp.float32),
                   jax.ShapeDtypeStruct((E, H), jnp.float32)],
    )(ef, gs, gd, we2e, be2e, wu, bu, we2n, be2n)


def _node_upd_body(nn, a0, a1, wu, bu, out):
    agg = a0[...] + a1[...]
    acc = jnp.dot(nn[...], wu[:H, :], preferred_element_type=jnp.float32)
    acc += jnp.dot(agg, wu[H:, :], preferred_element_type=jnp.float32)
    out[...] = jax.nn.relu(acc + bu[...])


def _node_update(nn, a0, a1, wu, bu):
    bn = 2000
    full = lambda a: pl.BlockSpec(a.shape, lambda i: (0,) * a.ndim)
    return pl.pallas_call(
        _node_upd_body,
        grid=(N // bn,),
        in_specs=[pl.BlockSpec((bn, H), lambda i: (i, 0)),
                  pl.BlockSpec((bn, H), lambda i: (i, 0)),
                  pl.BlockSpec((bn, H), lambda i: (i, 0)),
                  full(wu), full(bu)],
        out_specs=pl.BlockSpec((bn, H), lambda i: (i, 0)),
        out_shape=jax.ShapeDtypeStruct((N, H), jnp.float32),
    )(nn, a0, a1, wu, bu)


_RB = 1000


def _readout_body(nf, gid, aw, ab, p1, b1, p2, b2, out, hsum, hmax):
    i = pl.program_id(0)
    nblk = pl.num_programs(0)

    @pl.when(i == 0)
    def _():
        hsum[...] = jnp.zeros_like(hsum)
        hmax[...] = jnp.full_like(hmax, -jnp.inf)

    x = nf[...]
    ids = gid[0, 0, :]
    w = jax.nn.sigmoid(
        jnp.dot(x, aw[...], preferred_element_type=jnp.float32) + ab[...])
    wnf = w * x
    onehot = (lax.broadcasted_iota(jnp.int32, (G, _RB), 0)
              == ids[None, :]).astype(jnp.float32)
    hsum[...] += jnp.dot(onehot, wnf, preferred_element_type=jnp.float32)

    rows = []
    for g in range(G):
        m = jnp.where(ids == g, 0.0, -jnp.inf)
        rows.append(jnp.max(x + m[:, None], axis=0, keepdims=True))
    hmax[...] = jnp.maximum(hmax[...], jnp.concatenate(rows, axis=0))

    @pl.when(i == nblk - 1)
    def _():
        h1 = jnp.dot(hsum[...], p1[:H, :], preferred_element_type=jnp.float32)
        h1 += jnp.dot(hmax[...], p1[H:, :], preferred_element_type=jnp.float32)
        h1 += b1[...]
        out[...] = jnp.dot(h1, p2[...], preferred_element_type=jnp.float32) + b2[...]


def _readout(nf, gid3, aw, ab, p1, b1, p2, b2):
    full = lambda a: pl.BlockSpec(a.shape, lambda i: (0,) * a.ndim)
    return pl.pallas_call(
        _readout_body,
        grid=(N // _RB,),
        in_specs=[pl.BlockSpec((_RB, H), lambda i: (i, 0)),
                  pl.BlockSpec((1, 1, _RB), lambda i: (i, 0, 0)),
                  full(aw), full(ab), full(p1), full(b1), full(p2), full(b2)],
        out_specs=pl.BlockSpec((G, 1), lambda i: (0, 0)),
        out_shape=jax.ShapeDtypeStruct((G, 1), jnp.float32),
        scratch_shapes=[pltpu.VMEM((G, H), jnp.float32),
                        pltpu.VMEM((G, H), jnp.float32)],
    )(nf, gid3, aw, ab, p1, b1, p2, b2)


def kernel(node_feats, edge_feats, params, edge_index, graph_ids):
    src = edge_index[0]
    dst = edge_index[1]
    row = lambda v: v.reshape(1, -1)

    nf, ef = node_feats, edge_feats
    for p in params["layers"]:
        nn, cat = _node_proj(nf, p["w_n2n"], row(p["b_n2n"]),
                             p["w_l"], row(p["b_l"]),
                             p["w_r"], row(p["b_r"]))
        gs, gd = _gather_sc(cat, src, dst)
        new_ef, e2n = _edge_mm(ef, gs, gd,
                               p["w_e2e"], row(p["b_e2e"]),
                               p["w_ue"], row(p["b_ue"]),
                               p["w_e2n"], row(p["b_e2n"]))
        a0, a1 = _scatter_sc(e2n, dst)
        nf = _node_update(nn, a0, a1, p["w_un"], row(p["b_un"]))
        ef = new_ef

    gid3 = graph_ids.reshape(N // _RB, 1, _RB)
    return _readout(nf, gid3, params["atom_w"], row(params["atom_b"]),
                    params["p1_w"], row(params["p1_b"]),
                    params["p2_w"], row(params["p2_b"]))


# speedup vs baseline: 2.0448x; 2.0448x over previous
import functools

import jax
import jax.numpy as jnp
from jax import lax
from jax.experimental import pallas as pl
from jax.experimental.pallas import tpu as pltpu
from jax.experimental.pallas import tpu_sc as plsc

N = 10000
E = 320000
G = 64
H = 128

_NC = 2
_NS = 16
_NW = _NC * _NS
_PER_W = E // _NW
_CH = 80
_NCHUNK = _PER_W // _CH
_WB = 632
_WB_LAST = N - (_NS - 1) * _WB


def _node_proj_body(nf, w1, b1, wl, bl, wr, br, nn_out, cat_out):
    x = nf[...].astype(jnp.bfloat16)
    nn_out[...] = jax.nn.relu(
        jnp.dot(x, w1[...], preferred_element_type=jnp.float32) + b1[...])
    left = jnp.dot(x, wl[...], preferred_element_type=jnp.float32) + bl[...]
    right = jnp.dot(x, wr[...], preferred_element_type=jnp.float32) + br[...]
    lb = lax.bitcast_convert_type(left.astype(jnp.bfloat16),
                                  jnp.uint16).astype(jnp.uint32)
    rb = lax.bitcast_convert_type(right.astype(jnp.bfloat16),
                                  jnp.uint16).astype(jnp.uint32)
    cat_out[...] = lax.bitcast_convert_type(lb | (rb << 16), jnp.int32)


def _node_proj(nf, w1, b1, wl, bl, wr, br):
    bn = 2000
    nin = nf.shape[1]
    full = lambda a: pl.BlockSpec(a.shape, lambda i: (0,) * a.ndim)
    return pl.pallas_call(
        _node_proj_body,
        grid=(N // bn,),
        in_specs=[pl.BlockSpec((bn, nin), lambda i: (i, 0)),
                  full(w1), full(b1), full(wl), full(bl), full(wr), full(br)],
        out_specs=[pl.BlockSpec((bn, H), lambda i: (i, 0)),
                   pl.BlockSpec((bn, H), lambda i: (i, 0))],
        out_shape=[jax.ShapeDtypeStruct((N, H), jnp.float32),
                   jax.ShapeDtypeStruct((N, H), jnp.int32)],
    )(nf, w1, b1, wl, bl, wr, br)


def _gather_body(table, srcr, dstr, gs, gd, idx_s, idx_d, buf_s, buf_d,
                 sem_s, sem_d):
    wid = lax.axis_index("c") * _NS + lax.axis_index("s")
    base = wid * _PER_W

    def chunk(i, carry):
        off = base + i * _CH
        pltpu.sync_copy(srcr.at[pl.ds(off, _CH)], idx_s)
        pltpu.sync_copy(dstr.at[pl.ds(off, _CH)], idx_d)
        a = pltpu.async_copy(table.at[idx_s], buf_s, sem_s)
        b = pltpu.async_copy(table.at[idx_d], buf_d, sem_d)
        a.wait()
        b.wait()
        pltpu.sync_copy(buf_s, gs.at[pl.ds(off, _CH)])
        pltpu.sync_copy(buf_d, gd.at[pl.ds(off, _CH)])
        return carry

    lax.fori_loop(0, _NCHUNK, chunk, 0)


def _gather_sc(table, src, dst):
    mesh = plsc.VectorSubcoreMesh(core_axis_name="c", subcore_axis_name="s")
    k = functools.partial(
        pl.kernel,
        mesh=mesh,
        out_type=[jax.ShapeDtypeStruct((E, H), jnp.int32),
                  jax.ShapeDtypeStruct((E, H), jnp.int32)],
        scratch_types=[
            pltpu.VMEM((_CH,), jnp.int32),
            pltpu.VMEM((_CH,), jnp.int32),
            pltpu.VMEM((_CH, H), jnp.int32),
            pltpu.VMEM((_CH, H), jnp.int32),
            pltpu.SemaphoreType.DMA,
            pltpu.SemaphoreType.DMA,
        ],
    )(_gather_body)
    return k(table, src, dst)


def _scatter_body(e2n, dstr, zrows, out0, out1, idx, buf, agg, sem):
    cid = lax.axis_index("c")
    sid = lax.axis_index("s")
    zoff = pl.multiple_of(sid * _WB, 8)

    @pl.when(sid < _NS - 1)
    def _():
        pltpu.sync_copy(zrows, agg.at[pl.ds(zoff, _WB)])

    @pl.when(sid == _NS - 1)
    def _():
        pltpu.sync_copy(zrows.at[pl.ds(0, _WB_LAST)],
                        agg.at[pl.ds(zoff, _WB_LAST)])

    plsc.subcore_barrier()

    base = (cid * _NS + sid) * _PER_W

    def chunk(i, carry):
        off = base + i * _CH
        pltpu.sync_copy(e2n.at[pl.ds(off, _CH)], buf)
        pltpu.sync_copy(dstr.at[pl.ds(off, _CH)], idx)
        pltpu.sync_copy(buf, agg.at[idx], add=True)
        return carry

    lax.fori_loop(0, _NCHUNK, chunk, 0)
    plsc.subcore_barrier()

    @pl.when(cid == 0)
    def _():
        @pl.when(sid < _NS - 1)
        def _():
            pltpu.sync_copy(agg.at[pl.ds(zoff, _WB)], out0.at[pl.ds(zoff, _WB)])

        @pl.when(sid == _NS - 1)
        def _():
            pltpu.sync_copy(agg.at[pl.ds(zoff, _WB_LAST)],
                            out0.at[pl.ds(zoff, _WB_LAST)])

    @pl.when(cid == 1)
    def _():
        @pl.when(sid < _NS - 1)
        def _():
            pltpu.sync_copy(agg.at[pl.ds(zoff, _WB)], out1.at[pl.ds(zoff, _WB)])

        @pl.when(sid == _NS - 1)
        def _():
            pltpu.sync_copy(agg.at[pl.ds(zoff, _WB_LAST)],
                            out1.at[pl.ds(zoff, _WB_LAST)])


def _scatter_sc(e2n, dst):
    mesh = plsc.VectorSubcoreMesh(core_axis_name="c", subcore_axis_name="s")
    zrows = jnp.zeros((_WB, H), jnp.float32)
    k = functools.partial(
        pl.kernel,
        mesh=mesh,
        out_type=[jax.ShapeDtypeStruct((N, H), jnp.float32),
                  jax.ShapeDtypeStruct((N, H), jnp.float32)],
        scratch_types=[
            pltpu.VMEM((_CH,), jnp.int32),
            pltpu.VMEM((_CH, H), jnp.float32),
            pltpu.VMEM_SHARED((N, H), jnp.float32),
            pltpu.SemaphoreType.DMA,
        ],
    )(_scatter_body)
    return k(e2n, dst, zrows)


def _edge_body(ef, gs, gd, we2e, be2e, wu, bu, we2n, be2n, nef_out, e2n_out):
    x = ef[...].astype(jnp.bfloat16)
    unpack = lambda v: (
        lax.bitcast_convert_type(v << 16, jnp.float32),
        lax.bitcast_convert_type(v & jnp.int32(-65536), jnp.float32))
    ls, rs = unpack(gs[...])
    ld, rd = unpack(gd[...])
    first = jax.nn.relu(ls + rd).astype(jnp.bfloat16)
    second = jax.nn.relu(ld + rs).astype(jnp.bfloat16)
    third = jax.nn.relu(
        jnp.dot(x, we2e[...], preferred_element_type=jnp.float32) + be2e[...])
    acc = jnp.dot(first, wu[:H, :], preferred_element_type=jnp.float32)
    acc += jnp.dot(second, wu[H:2 * H, :], preferred_element_type=jnp.float32)
    acc += jnp.dot(third.astype(jnp.bfloat16), wu[2 * H:, :],
                   preferred_element_type=jnp.float32)
    nef_out[...] = jax.nn.relu(acc + bu[...]).astype(jnp.bfloat16)
    e2n_out[...] = jax.nn.relu(
        jnp.dot(x, we2n[...], preferred_element_type=jnp.float32) + be2n[...])


def _edge_mm(ef, gs, gd, we2e, be2e, wu, bu, we2n, be2n):
    be = 2000
    ein = ef.shape[1]
    full = lambda a: pl.BlockSpec(a.shape, lambda i: (0,) * a.ndim)
    return pl.pallas_call(
        _edge_body,
        grid=(E // be,),
        in_specs=[pl.BlockSpec((be, ein), lambda i: (i, 0)),
                  pl.BlockSpec((be, H), lambda i: (i, 0)),
                  pl.BlockSpec((be, H), lambda i: (i, 0)),
                  full(we2e), full(be2e), full(wu), full(bu),
                  full(we2n), full(be2n)],
        out_specs=[pl.BlockSpec((be, H), lambda i: (i, 0)),
                   pl.BlockSpec((be, H), lambda i: (i, 0))],
        out_shape=[jax.ShapeDtypeStruct((E, H), jnp.bfloat16),
                   jax.ShapeDtypeStruct((E, H), jnp.float32)],
    )(ef, gs, gd, we2e, be2e, wu, bu, we2n, be2n)


def _node_upd_body(nn, a0, a1, wu, bu, out):
    agg = (a0[...] + a1[...]).astype(jnp.bfloat16)
    acc = jnp.dot(nn[...].astype(jnp.bfloat16), wu[:H, :],
                  preferred_element_type=jnp.float32)
    acc += jnp.dot(agg, wu[H:, :], preferred_element_type=jnp.float32)
    out[...] = jax.nn.relu(acc + bu[...])


def _node_update(nn, a0, a1, wu, bu):
    bn = 2000
    full = lambda a: pl.BlockSpec(a.shape, lambda i: (0,) * a.ndim)
    return pl.pallas_call(
        _node_upd_body,
        grid=(N // bn,),
        in_specs=[pl.BlockSpec((bn, H), lambda i: (i, 0)),
                  pl.BlockSpec((bn, H), lambda i: (i, 0)),
                  pl.BlockSpec((bn, H), lambda i: (i, 0)),
                  full(wu), full(bu)],
        out_specs=pl.BlockSpec((bn, H), lambda i: (i, 0)),
        out_shape=jax.ShapeDtypeStruct((N, H), jnp.float32),
    )(nn, a0, a1, wu, bu)


_RB = 1000


def _readout_body(nf, gid, aw, ab, p1, b1, p2, b2, out, hsum, hmax):
    i = pl.program_id(0)
    nblk = pl.num_programs(0)

    @pl.when(i == 0)
    def _():
        hsum[...] = jnp.zeros_like(hsum)
        hmax[...] = jnp.full_like(hmax, -jnp.inf)

    x = nf[...]
    ids = gid[0, 0, :]
    w = jax.nn.sigmoid(
        jnp.dot(x, aw[...], preferred_element_type=jnp.float32) + ab[...])
    wnf = w * x
    onehot = (lax.broadcasted_iota(jnp.int32, (G, _RB), 0)
              == ids[None, :]).astype(jnp.float32)
    hsum[...] += jnp.dot(onehot, wnf, preferred_element_type=jnp.float32)

    rows = []
    for g in range(G):
        m = jnp.where(ids == g, 0.0, -jnp.inf)
        rows.append(jnp.max(x + m[:, None], axis=0, keepdims=True))
    hmax[...] = jnp.maximum(hmax[...], jnp.concatenate(rows, axis=0))

    @pl.when(i == nblk - 1)
    def _():
        h1 = jnp.dot(hsum[...], p1[:H, :], preferred_element_type=jnp.float32)
        h1 += jnp.dot(hmax[...], p1[H:, :], preferred_element_type=jnp.float32)
        h1 += b1[...]
        out[...] = jnp.dot(h1, p2[...], preferred_element_type=jnp.float32) + b2[...]


def _readout(nf, gid3, aw, ab, p1, b1, p2, b2):
    full = lambda a: pl.BlockSpec(a.shape, lambda i: (0,) * a.ndim)
    return pl.pallas_call(
        _readout_body,
        grid=(N // _RB,),
        in_specs=[pl.BlockSpec((_RB, H), lambda i: (i, 0)),
                  pl.BlockSpec((1, 1, _RB), lambda i: (i, 0, 0)),
                  full(aw), full(ab), full(p1), full(b1), full(p2), full(b2)],
        out_specs=pl.BlockSpec((G, 1), lambda i: (0, 0)),
        out_shape=jax.ShapeDtypeStruct((G, 1), jnp.float32),
        scratch_shapes=[pltpu.VMEM((G, H), jnp.float32),
                        pltpu.VMEM((G, H), jnp.float32)],
    )(nf, gid3, aw, ab, p1, b1, p2, b2)


def kernel(node_feats, edge_feats, params, edge_index, graph_ids):
    src = edge_index[0]
    dst = edge_index[1]
    row = lambda v: v.reshape(1, -1)

    bf = lambda w: w.astype(jnp.bfloat16)

    nf, ef = node_feats, edge_feats
    for p in params["layers"]:
        nn, cat = _node_proj(nf, bf(p["w_n2n"]), row(p["b_n2n"]),
                             bf(p["w_l"]), row(p["b_l"]),
                             bf(p["w_r"]), row(p["b_r"]))
        gs, gd = _gather_sc(cat, src, dst)
        new_ef, e2n = _edge_mm(ef, gs, gd,
                               bf(p["w_e2e"]), row(p["b_e2e"]),
                               bf(p["w_ue"]), row(p["b_ue"]),
                               bf(p["w_e2n"]), row(p["b_e2n"]))
        a0, a1 = _scatter_sc(e2n, dst)
        nf = _node_update(nn, a0, a1, bf(p["w_un"]), row(p["b_un"]))
        ef = new_ef

    gid3 = graph_ids.reshape(N // _RB, 1, _RB)
    return _readout(nf, gid3, params["atom_w"], row(params["atom_b"]),
                    params["p1_w"], row(params["p1_b"]),
                    params["p2_w"], row(params["p2_b"]))


# speedup vs baseline: 2.6628x; 1.3022x over previous
import functools

import jax
import jax.numpy as jnp
from jax import lax
from jax.experimental import pallas as pl
from jax.experimental.pallas import tpu as pltpu
from jax.experimental.pallas import tpu_sc as plsc

N = 10000
E = 320000
G = 64
H = 128

_NC = 2
_NS = 16
_NW = _NC * _NS
_PER_W = E // _NW
_CH = 40
_NCHUNK = _PER_W // _CH
_NB = 5
_PASSES = _NCHUNK // _NB
_CHS = _CH
_NCHUNK_S = _NCHUNK
_NBS = 2
_PASSES_S = _NCHUNK_S // _NBS
_WB = 632
_WB_LAST = N - (_NS - 1) * _WB


def _node_proj_body(nf, w1, b1, wl, bl, wr, br, nn_out, cat_out):
    x = nf[...]
    nn_out[...] = jax.nn.relu(
        jnp.dot(x, w1[...], preferred_element_type=jnp.float32) + b1[...])
    left = jnp.dot(x, wl[...], preferred_element_type=jnp.float32) + bl[...]
    right = jnp.dot(x, wr[...], preferred_element_type=jnp.float32) + br[...]
    lb = lax.bitcast_convert_type(left.astype(jnp.bfloat16),
                                  jnp.uint16).astype(jnp.uint32)
    rb = lax.bitcast_convert_type(right.astype(jnp.bfloat16),
                                  jnp.uint16).astype(jnp.uint32)
    cat_out[...] = lax.bitcast_convert_type(lb | (rb << 16), jnp.int32)


def _node_proj(nf, w1, b1, wl, bl, wr, br):
    bn = 2000
    nin = nf.shape[1]
    full = lambda a: pl.BlockSpec(a.shape, lambda i: (0,) * a.ndim)
    return pl.pallas_call(
        _node_proj_body,
        grid=(N // bn,),
        in_specs=[pl.BlockSpec((bn, nin), lambda i: (i, 0)),
                  full(w1), full(b1), full(wl), full(bl), full(wr), full(br)],
        out_specs=[pl.BlockSpec((bn, H), lambda i: (i, 0)),
                   pl.BlockSpec((bn, H), lambda i: (i, 0))],
        out_shape=[jax.ShapeDtypeStruct((N, H), jnp.float32),
                   jax.ShapeDtypeStruct((N, H), jnp.int32)],
    )(nf, w1, b1, wl, bl, wr, br)


def _gather_body(table, src3, dst3, gs, gd, *rest):
    idxs, idxd = rest[0], rest[1]
    bufs_s = rest[2:2 + _NB]
    bufs_d = rest[2 + _NB:2 + 2 * _NB]
    semg = rest[2 + 2 * _NB:2 + 3 * _NB]
    semw = rest[2 + 3 * _NB:2 + 4 * _NB]

    wid = lax.axis_index("c") * _NS + lax.axis_index("s")
    base = wid * _PER_W
    pltpu.sync_copy(src3.at[pl.ds(wid, 1)], idxs)
    pltpu.sync_copy(dst3.at[pl.ds(wid, 1)], idxd)

    def gpass(j, carry):
        for b in range(_NB):
            c = j * _NB + b

            @pl.when(j > 0)
            def _():
                po = base + (c - _NB) * _CH
                pltpu.make_async_copy(
                    bufs_s[b], gs.at[pl.ds(po, _CH)], semw[b]).wait()
                pltpu.make_async_copy(
                    bufs_d[b], gd.at[pl.ds(po, _CH)], semw[b]).wait()

            pltpu.async_copy(table.at[idxs.at[0, c]], bufs_s[b], semg[b])
            pltpu.async_copy(table.at[idxd.at[0, c]], bufs_d[b], semg[b])
        for b in range(_NB):
            c = j * _NB + b
            lo = base + c * _CH
            pltpu.make_async_copy(
                table.at[idxs.at[0, c]], bufs_s[b], semg[b]).wait()
            pltpu.make_async_copy(
                table.at[idxd.at[0, c]], bufs_d[b], semg[b]).wait()
            pltpu.async_copy(bufs_s[b], gs.at[pl.ds(lo, _CH)], semw[b])
            pltpu.async_copy(bufs_d[b], gd.at[pl.ds(lo, _CH)], semw[b])
        return carry

    lax.fori_loop(0, _PASSES, gpass, 0)
    for b in range(_NB):
        lo = base + ((_PASSES - 1) * _NB + b) * _CH
        pltpu.make_async_copy(bufs_s[b], gs.at[pl.ds(lo, _CH)], semw[b]).wait()
        pltpu.make_async_copy(bufs_d[b], gd.at[pl.ds(lo, _CH)], semw[b]).wait()


def _gather_sc(table, src3, dst3):
    mesh = plsc.VectorSubcoreMesh(core_axis_name="c", subcore_axis_name="s")
    k = functools.partial(
        pl.kernel,
        mesh=mesh,
        out_type=[jax.ShapeDtypeStruct((E, H), jnp.int32),
                  jax.ShapeDtypeStruct((E, H), jnp.int32)],
        scratch_types=(
            [pltpu.VMEM((1, _NCHUNK, _CH), jnp.int32)] * 2
            + [pltpu.VMEM((_CH, H), jnp.int32)] * (2 * _NB)
            + [pltpu.SemaphoreType.DMA] * (2 * _NB)
        ),
    )(_gather_body)
    return k(table, src3, dst3)


def _scatter_body(e2n, dst3, zrows, out0, out1, *rest):
    idxd = rest[0]
    bufs = rest[1:1 + _NBS]
    seml = rest[1 + _NBS:1 + 2 * _NBS]
    sema = rest[1 + 2 * _NBS:1 + 3 * _NBS]
    agg = rest[1 + 3 * _NBS]

    cid = lax.axis_index("c")
    sid = lax.axis_index("s")
    wid = cid * _NS + sid
    base = wid * _PER_W
    pltpu.sync_copy(dst3.at[pl.ds(wid, 1)], idxd)

    zoff = pl.multiple_of(sid * _WB, 8)

    @pl.when(sid < _NS - 1)
    def _():
        pltpu.sync_copy(zrows, agg.at[pl.ds(zoff, _WB)])

    @pl.when(sid == _NS - 1)
    def _():
        pltpu.sync_copy(zrows.at[pl.ds(0, _WB_LAST)],
                        agg.at[pl.ds(zoff, _WB_LAST)])

    plsc.subcore_barrier()

    def spass(j, carry):
        for b in range(_NBS):
            c = j * _NBS + b

            @pl.when(j > 0)
            def _():
                pltpu.make_async_copy(
                    bufs[b], agg.at[idxd.at[0, c]], sema[b]).wait()

            pltpu.async_copy(e2n.at[pl.ds(base + c * _CHS, _CHS)], bufs[b],
                             seml[b])
        for b in range(_NBS):
            c = j * _NBS + b
            pltpu.make_async_copy(
                e2n.at[pl.ds(base + c * _CHS, _CHS)], bufs[b], seml[b]).wait()
            pltpu.async_copy(bufs[b], agg.at[idxd.at[0, c]], sema[b],
                             add=True)
        return carry

    lax.fori_loop(0, _PASSES_S, spass, 0)
    for b in range(_NBS):
        pltpu.make_async_copy(bufs[b], agg.at[idxd.at[0, 0]], sema[b]).wait()
    plsc.subcore_barrier()

    @pl.when(cid == 0)
    def _():
        @pl.when(sid < _NS - 1)
        def _():
            pltpu.sync_copy(agg.at[pl.ds(zoff, _WB)], out0.at[pl.ds(zoff, _WB)])

        @pl.when(sid == _NS - 1)
        def _():
            pltpu.sync_copy(agg.at[pl.ds(zoff, _WB_LAST)],
                            out0.at[pl.ds(zoff, _WB_LAST)])

    @pl.when(cid == 1)
    def _():
        @pl.when(sid < _NS - 1)
        def _():
            pltpu.sync_copy(agg.at[pl.ds(zoff, _WB)], out1.at[pl.ds(zoff, _WB)])

        @pl.when(sid == _NS - 1)
        def _():
            pltpu.sync_copy(agg.at[pl.ds(zoff, _WB_LAST)],
                            out1.at[pl.ds(zoff, _WB_LAST)])


def _scatter_sc(e2n, dst3):
    mesh = plsc.VectorSubcoreMesh(core_axis_name="c", subcore_axis_name="s")
    zrows = jnp.zeros((_WB, H), jnp.float32)
    k = functools.partial(
        pl.kernel,
        mesh=mesh,
        out_type=[jax.ShapeDtypeStruct((N, H), jnp.float32),
                  jax.ShapeDtypeStruct((N, H), jnp.float32)],
        scratch_types=(
            [pltpu.VMEM((1, _NCHUNK_S, _CHS), jnp.int32)]
            + [pltpu.VMEM((_CHS, H), jnp.float32)] * _NBS
            + [pltpu.SemaphoreType.DMA] * (2 * _NBS)
            + [pltpu.VMEM_SHARED((N, H), jnp.float32)]
        ),
    )(_scatter_body)
    return k(e2n, dst3, zrows)


def _edge_body(ef, gs, gd, we2e, be2e, wu, bu, we2n, be2n, nef_out, e2n_out):
    x = ef[...]
    unpack = lambda v: (
        lax.bitcast_convert_type(v << 16, jnp.float32),
        lax.bitcast_convert_type(v & jnp.int32(-65536), jnp.float32))
    ls, rs = unpack(gs[...])
    ld, rd = unpack(gd[...])
    first = jax.nn.relu(ls + rd)
    second = jax.nn.relu(ld + rs)
    third = jax.nn.relu(
        jnp.dot(x, we2e[...], preferred_element_type=jnp.float32) + be2e[...])
    acc = jnp.dot(first, wu[:H, :], preferred_element_type=jnp.float32)
    acc += jnp.dot(second, wu[H:2 * H, :], preferred_element_type=jnp.float32)
    acc += jnp.dot(third, wu[2 * H:, :], preferred_element_type=jnp.float32)
    nef_out[...] = jax.nn.relu(acc + bu[...])
    e2n_out[...] = jax.nn.relu(
        jnp.dot(x, we2n[...], preferred_element_type=jnp.float32) + be2n[...])


def _edge_mm(ef, gs, gd, we2e, be2e, wu, bu, we2n, be2n):
    be = 2000
    ein = ef.shape[1]
    full = lambda a: pl.BlockSpec(a.shape, lambda i: (0,) * a.ndim)
    return pl.pallas_call(
        _edge_body,
        grid=(E // be,),
        in_specs=[pl.BlockSpec((be, ein), lambda i: (i, 0)),
                  pl.BlockSpec((be, H), lambda i: (i, 0)),
                  pl.BlockSpec((be, H), lambda i: (i, 0)),
                  full(we2e), full(be2e), full(wu), full(bu),
                  full(we2n), full(be2n)],
        out_specs=[pl.BlockSpec((be, H), lambda i: (i, 0)),
                   pl.BlockSpec((be, H), lambda i: (i, 0))],
        out_shape=[jax.ShapeDtypeStruct((E, H), jnp.float32),
                   jax.ShapeDtypeStruct((E, H), jnp.float32)],
    )(ef, gs, gd, we2e, be2e, wu, bu, we2n, be2n)


def _node_upd_body(nn, a0, a1, wu, bu, out):
    agg = a0[...] + a1[...]
    acc = jnp.dot(nn[...], wu[:H, :], preferred_element_type=jnp.float32)
    acc += jnp.dot(agg, wu[H:, :], preferred_element_type=jnp.float32)
    out[...] = jax.nn.relu(acc + bu[...])


def _node_update(nn, a0, a1, wu, bu):
    bn = 2000
    full = lambda a: pl.BlockSpec(a.shape, lambda i: (0,) * a.ndim)
    return pl.pallas_call(
        _node_upd_body,
        grid=(N // bn,),
        in_specs=[pl.BlockSpec((bn, H), lambda i: (i, 0)),
                  pl.BlockSpec((bn, H), lambda i: (i, 0)),
                  pl.BlockSpec((bn, H), lambda i: (i, 0)),
                  full(wu), full(bu)],
        out_specs=pl.BlockSpec((bn, H), lambda i: (i, 0)),
        out_shape=jax.ShapeDtypeStruct((N, H), jnp.float32),
    )(nn, a0, a1, wu, bu)


_RB = 1000


def _readout_body(nf, gid, aw, ab, p1, b1, p2, b2, out, hsum, hmax):
    i = pl.program_id(0)
    nblk = pl.num_programs(0)

    @pl.when(i == 0)
    def _():
        hsum[...] = jnp.zeros_like(hsum)
        hmax[...] = jnp.full_like(hmax, -jnp.inf)

    x = nf[...]
    ids = gid[0, 0, :]
    w = jax.nn.sigmoid(
        jnp.dot(x, aw[...], preferred_element_type=jnp.float32) + ab[...])
    wnf = w * x
    onehot = (lax.broadcasted_iota(jnp.int32, (G, _RB), 0)
              == ids[None, :]).astype(jnp.float32)
    hsum[...] += jnp.dot(onehot, wnf, preferred_element_type=jnp.float32)

    rows = []
    for g in range(G):
        m = jnp.where(ids == g, 0.0, -jnp.inf)
        rows.append(jnp.max(x + m[:, None], axis=0, keepdims=True))
    hmax[...] = jnp.maximum(hmax[...], jnp.concatenate(rows, axis=0))

    @pl.when(i == nblk - 1)
    def _():
        h1 = jnp.dot(hsum[...], p1[:H, :], preferred_element_type=jnp.float32)
        h1 += jnp.dot(hmax[...], p1[H:, :], preferred_element_type=jnp.float32)
        h1 += b1[...]
        out[...] = jnp.dot(h1, p2[...], preferred_element_type=jnp.float32) + b2[...]


def _readout(nf, gid3, aw, ab, p1, b1, p2, b2):
    full = lambda a: pl.BlockSpec(a.shape, lambda i: (0,) * a.ndim)
    return pl.pallas_call(
        _readout_body,
        grid=(N // _RB,),
        in_specs=[pl.BlockSpec((_RB, H), lambda i: (i, 0)),
                  pl.BlockSpec((1, 1, _RB), lambda i: (i, 0, 0)),
                  full(aw), full(ab), full(p1), full(b1), full(p2), full(b2)],
        out_specs=pl.BlockSpec((G, 1), lambda i: (0, 0)),
        out_shape=jax.ShapeDtypeStruct((G, 1), jnp.float32),
        scratch_shapes=[pltpu.VMEM((G, H), jnp.float32),
                        pltpu.VMEM((G, H), jnp.float32)],
    )(nf, gid3, aw, ab, p1, b1, p2, b2)


def kernel(node_feats, edge_feats, params, edge_index, graph_ids):
    src3 = edge_index[0].reshape(_NW, _NCHUNK, _CH)
    dst3 = edge_index[1].reshape(_NW, _NCHUNK, _CH)
    dst3s = edge_index[1].reshape(_NW, _NCHUNK_S, _CHS)
    row = lambda v: v.reshape(1, -1)

    nf, ef = node_feats, edge_feats
    for p in params["layers"]:
        nn, cat = _node_proj(nf, p["w_n2n"], row(p["b_n2n"]),
                             p["w_l"], row(p["b_l"]),
                             p["w_r"], row(p["b_r"]))
        gs, gd = _gather_sc(cat, src3, dst3)
        new_ef, e2n = _edge_mm(ef, gs, gd,
                               p["w_e2e"], row(p["b_e2e"]),
                               p["w_ue"], row(p["b_ue"]),
                               p["w_e2n"], row(p["b_e2n"]))
        a0, a1 = _scatter_sc(e2n, dst3s)
        nf = _node_update(nn, a0, a1, p["w_un"], row(p["b_un"]))
        ef = new_ef

    gid3 = graph_ids.reshape(N // _RB, 1, _RB)
    return _readout(nf, gid3, params["atom_w"], row(params["atom_b"]),
                    params["p1_w"], row(params["p1_b"]),
                    params["p2_w"], row(params["p2_b"]))


# speedup vs baseline: 3.8694x; 1.4531x over previous
import functools

import jax
import jax.numpy as jnp
from jax import lax
from jax.experimental import pallas as pl
from jax.experimental.pallas import tpu as pltpu
from jax.experimental.pallas import tpu_sc as plsc

N = 10000
E = 320000
G = 64
H = 128

_NC = 2
_NS = 16
_NW = _NC * _NS
_PER_W = E // _NW
_CH = 40
_NCHUNK = _PER_W // _CH
_NB = 5
_PASSES = _NCHUNK // _NB
_CHS = _CH
_NCHUNK_S = _NCHUNK
_NBS = 2
_PASSES_S = _NCHUNK_S // _NBS
_WB = 632
_WB_LAST = N - (_NS - 1) * _WB


def _node_proj_body(nf, w1, b1, wl, bl, wr, br, nn_out, cat_out):
    x = nf[...]
    nn_out[...] = jax.nn.relu(
        jnp.dot(x, w1[...], preferred_element_type=jnp.float32) + b1[...])
    left = jnp.dot(x, wl[...], preferred_element_type=jnp.float32) + bl[...]
    right = jnp.dot(x, wr[...], preferred_element_type=jnp.float32) + br[...]
    lb = lax.bitcast_convert_type(left.astype(jnp.bfloat16),
                                  jnp.uint16).astype(jnp.uint32)
    rb = lax.bitcast_convert_type(right.astype(jnp.bfloat16),
                                  jnp.uint16).astype(jnp.uint32)
    cat_out[...] = lax.bitcast_convert_type(lb | (rb << 16), jnp.int32)


def _node_proj(nf, w1, b1, wl, bl, wr, br):
    bn = 2000
    nin = nf.shape[1]
    full = lambda a: pl.BlockSpec(a.shape, lambda i: (0,) * a.ndim)
    return pl.pallas_call(
        _node_proj_body,
        grid=(N // bn,),
        in_specs=[pl.BlockSpec((bn, nin), lambda i: (i, 0)),
                  full(w1), full(b1), full(wl), full(bl), full(wr), full(br)],
        out_specs=[pl.BlockSpec((bn, H), lambda i: (i, 0)),
                   pl.BlockSpec((bn, H), lambda i: (i, 0))],
        out_shape=[jax.ShapeDtypeStruct((N, H), jnp.float32),
                   jax.ShapeDtypeStruct((N, H), jnp.int32)],
    )(nf, w1, b1, wl, bl, wr, br)


def _gather_body(table, src3, dst3, gs, gd, *rest):
    idxs, idxd = rest[0], rest[1]
    bufs_s = rest[2:2 + _NB]
    bufs_d = rest[2 + _NB:2 + 2 * _NB]
    semg = rest[2 + 2 * _NB:2 + 3 * _NB]
    semw = rest[2 + 3 * _NB:2 + 4 * _NB]

    wid = lax.axis_index("c") * _NS + lax.axis_index("s")
    base = wid * _PER_W
    pltpu.sync_copy(src3.at[pl.ds(wid, 1)], idxs)
    pltpu.sync_copy(dst3.at[pl.ds(wid, 1)], idxd)

    def gpass(j, carry):
        for b in range(_NB):
            c = j * _NB + b

            @pl.when(j > 0)
            def _():
                po = base + (c - _NB) * _CH
                pltpu.make_async_copy(
                    bufs_s[b], gs.at[pl.ds(po, _CH)], semw[b]).wait()
                pltpu.make_async_copy(
                    bufs_d[b], gd.at[pl.ds(po, _CH)], semw[b]).wait()

            pltpu.async_copy(table.at[idxs.at[0, c]], bufs_s[b], semg[b])
            pltpu.async_copy(table.at[idxd.at[0, c]], bufs_d[b], semg[b])
        for b in range(_NB):
            c = j * _NB + b
            lo = base + c * _CH
            pltpu.make_async_copy(
                table.at[idxs.at[0, c]], bufs_s[b], semg[b]).wait()
            pltpu.make_async_copy(
                table.at[idxd.at[0, c]], bufs_d[b], semg[b]).wait()
            pltpu.async_copy(bufs_s[b], gs.at[pl.ds(lo, _CH)], semw[b])
            pltpu.async_copy(bufs_d[b], gd.at[pl.ds(lo, _CH)], semw[b])
        return carry

    lax.fori_loop(0, _PASSES, gpass, 0)
    for b in range(_NB):
        lo = base + ((_PASSES - 1) * _NB + b) * _CH
        pltpu.make_async_copy(bufs_s[b], gs.at[pl.ds(lo, _CH)], semw[b]).wait()
        pltpu.make_async_copy(bufs_d[b], gd.at[pl.ds(lo, _CH)], semw[b]).wait()


def _gather_sc(table, src3, dst3):
    mesh = plsc.VectorSubcoreMesh(core_axis_name="c", subcore_axis_name="s")
    k = functools.partial(
        pl.kernel,
        mesh=mesh,
        out_type=[jax.ShapeDtypeStruct((E, H), jnp.int32),
                  jax.ShapeDtypeStruct((E, H), jnp.int32)],
        scratch_types=(
            [pltpu.VMEM((1, _NCHUNK, _CH), jnp.int32)] * 2
            + [pltpu.VMEM((_CH, H), jnp.int32)] * (2 * _NB)
            + [pltpu.SemaphoreType.DMA] * (2 * _NB)
        ),
    )(_gather_body)
    return k(table, src3, dst3)


def _scatter_body(e2n, dst3, zrows, out0, out1, *rest):
    idxd = rest[0]
    bufs = rest[1:1 + _NBS]
    seml = rest[1 + _NBS:1 + 2 * _NBS]
    sema = rest[1 + 2 * _NBS:1 + 3 * _NBS]
    agg = rest[1 + 3 * _NBS]

    cid = lax.axis_index("c")
    sid = lax.axis_index("s")
    wid = cid * _NS + sid
    base = wid * _PER_W
    pltpu.sync_copy(dst3.at[pl.ds(wid, 1)], idxd)

    zoff = pl.multiple_of(sid * _WB, 8)

    @pl.when(sid < _NS - 1)
    def _():
        pltpu.sync_copy(zrows, agg.at[pl.ds(zoff, _WB)])

    @pl.when(sid == _NS - 1)
    def _():
        pltpu.sync_copy(zrows.at[pl.ds(0, _WB_LAST)],
                        agg.at[pl.ds(zoff, _WB_LAST)])

    plsc.subcore_barrier()

    def spass(j, carry):
        for b in range(_NBS):
            c = j * _NBS + b

            @pl.when(j > 0)
            def _():
                pltpu.make_async_copy(
                    bufs[b], agg.at[idxd.at[0, c]], sema[b]).wait()

            pltpu.async_copy(e2n.at[pl.ds(base + c * _CHS, _CHS)], bufs[b],
                             seml[b])
        for b in range(_NBS):
            c = j * _NBS + b
            pltpu.make_async_copy(
                e2n.at[pl.ds(base + c * _CHS, _CHS)], bufs[b], seml[b]).wait()
            pltpu.async_copy(bufs[b], agg.at[idxd.at[0, c]], sema[b],
                             add=True)
        return carry

    lax.fori_loop(0, _PASSES_S, spass, 0)
    for b in range(_NBS):
        pltpu.make_async_copy(bufs[b], agg.at[idxd.at[0, 0]], sema[b]).wait()
    plsc.subcore_barrier()

    @pl.when(cid == 0)
    def _():
        @pl.when(sid < _NS - 1)
        def _():
            pltpu.sync_copy(agg.at[pl.ds(zoff, _WB)], out0.at[pl.ds(zoff, _WB)])

        @pl.when(sid == _NS - 1)
        def _():
            pltpu.sync_copy(agg.at[pl.ds(zoff, _WB_LAST)],
                            out0.at[pl.ds(zoff, _WB_LAST)])

    @pl.when(cid == 1)
    def _():
        @pl.when(sid < _NS - 1)
        def _():
            pltpu.sync_copy(agg.at[pl.ds(zoff, _WB)], out1.at[pl.ds(zoff, _WB)])

        @pl.when(sid == _NS - 1)
        def _():
            pltpu.sync_copy(agg.at[pl.ds(zoff, _WB_LAST)],
                            out1.at[pl.ds(zoff, _WB_LAST)])


def _scatter_sc(e2n, dst3):
    mesh = plsc.VectorSubcoreMesh(core_axis_name="c", subcore_axis_name="s")
    zrows = jnp.zeros((_WB, H), jnp.float32)
    k = functools.partial(
        pl.kernel,
        mesh=mesh,
        out_type=[jax.ShapeDtypeStruct((N, H), jnp.float32),
                  jax.ShapeDtypeStruct((N, H), jnp.float32)],
        scratch_types=(
            [pltpu.VMEM((1, _NCHUNK_S, _CHS), jnp.int32)]
            + [pltpu.VMEM((_CHS, H), jnp.float32)] * _NBS
            + [pltpu.SemaphoreType.DMA] * (2 * _NBS)
            + [pltpu.VMEM_SHARED((N, H), jnp.float32)]
        ),
    )(_scatter_body)
    return k(e2n, dst3, zrows)


def _msg_body(ef, we2n, be2n, e2n_out):
    e2n_out[...] = jax.nn.relu(
        jnp.dot(ef[...], we2n[...], preferred_element_type=jnp.float32)
        + be2n[...])


def _msg_mm(ef, we2n, be2n):
    be = 2000
    ein = ef.shape[1]
    full = lambda a: pl.BlockSpec(a.shape, lambda i: (0,) * a.ndim)
    return pl.pallas_call(
        _msg_body,
        grid=(E // be,),
        in_specs=[pl.BlockSpec((be, ein), lambda i: (i, 0)),
                  full(we2n), full(be2n)],
        out_specs=pl.BlockSpec((be, H), lambda i: (i, 0)),
        out_shape=jax.ShapeDtypeStruct((E, H), jnp.float32),
    )(ef, we2n, be2n)


def _ue_body(ef, gs, gd, we2e, be2e, wu, bu, nef_out):
    x = ef[...]
    unpack = lambda v: (
        lax.bitcast_convert_type(v << 16, jnp.float32),
        lax.bitcast_convert_type(v & jnp.int32(-65536), jnp.float32))
    ls, rs = unpack(gs[...])
    ld, rd = unpack(gd[...])
    first = jax.nn.relu(ls + rd)
    second = jax.nn.relu(ld + rs)
    third = jax.nn.relu(
        jnp.dot(x, we2e[...], preferred_element_type=jnp.float32) + be2e[...])
    acc = jnp.dot(first, wu[:H, :], preferred_element_type=jnp.float32)
    acc += jnp.dot(second, wu[H:2 * H, :], preferred_element_type=jnp.float32)
    acc += jnp.dot(third, wu[2 * H:, :], preferred_element_type=jnp.float32)
    nef_out[...] = jax.nn.relu(acc + bu[...])


def _ue_mm(ef, gs, gd, we2e, be2e, wu, bu):
    be = 2000
    ein = ef.shape[1]
    full = lambda a: pl.BlockSpec(a.shape, lambda i: (0,) * a.ndim)
    return pl.pallas_call(
        _ue_body,
        grid=(E // be,),
        in_specs=[pl.BlockSpec((be, ein), lambda i: (i, 0)),
                  pl.BlockSpec((be, H), lambda i: (i, 0)),
                  pl.BlockSpec((be, H), lambda i: (i, 0)),
                  full(we2e), full(be2e), full(wu), full(bu)],
        out_specs=pl.BlockSpec((be, H), lambda i: (i, 0)),
        out_shape=jax.ShapeDtypeStruct((E, H), jnp.float32),
    )(ef, gs, gd, we2e, be2e, wu, bu)


def _node_upd_body(nn, a0, a1, wu, bu, out):
    agg = a0[...] + a1[...]
    acc = jnp.dot(nn[...], wu[:H, :], preferred_element_type=jnp.float32)
    acc += jnp.dot(agg, wu[H:, :], preferred_element_type=jnp.float32)
    out[...] = jax.nn.relu(acc + bu[...])


def _node_update(nn, a0, a1, wu, bu):
    bn = 2000
    full = lambda a: pl.BlockSpec(a.shape, lambda i: (0,) * a.ndim)
    return pl.pallas_call(
        _node_upd_body,
        grid=(N // bn,),
        in_specs=[pl.BlockSpec((bn, H), lambda i: (i, 0)),
                  pl.BlockSpec((bn, H), lambda i: (i, 0)),
                  pl.BlockSpec((bn, H), lambda i: (i, 0)),
                  full(wu), full(bu)],
        out_specs=pl.BlockSpec((bn, H), lambda i: (i, 0)),
        out_shape=jax.ShapeDtypeStruct((N, H), jnp.float32),
    )(nn, a0, a1, wu, bu)


_RB = 1000


def _readout_body(nf, gid, aw, ab, p1, b1, p2, b2, out, hsum, hmax):
    i = pl.program_id(0)
    nblk = pl.num_programs(0)

    @pl.when(i == 0)
    def _():
        hsum[...] = jnp.zeros_like(hsum)
        hmax[...] = jnp.full_like(hmax, -jnp.inf)

    x = nf[...]
    ids = gid[0, 0, :]
    w = jax.nn.sigmoid(
        jnp.dot(x, aw[...], preferred_element_type=jnp.float32) + ab[...])
    wnf = w * x
    onehot = (lax.broadcasted_iota(jnp.int32, (G, _RB), 0)
              == ids[None, :]).astype(jnp.float32)
    hsum[...] += jnp.dot(onehot, wnf, preferred_element_type=jnp.float32)

    lo = gid[0, 0, 0]
    hi = gid[0, 0, _RB - 1]
    for g in range(G):
        @pl.when(jnp.logical_and(lo <= g, g <= hi))
        def _(g=g):
            m = jnp.where(ids == g, 0.0, -jnp.inf)
            rowmax = jnp.max(x + m[:, None], axis=0, keepdims=True)
            hmax[g:g + 1, :] = jnp.maximum(hmax[g:g + 1, :], rowmax)

    @pl.when(i == nblk - 1)
    def _():
        h1 = jnp.dot(hsum[...], p1[:H, :], preferred_element_type=jnp.float32)
        h1 += jnp.dot(hmax[...], p1[H:, :], preferred_element_type=jnp.float32)
        h1 += b1[...]
        out[...] = jnp.dot(h1, p2[...], preferred_element_type=jnp.float32) + b2[...]


def _readout(nf, gid3, aw, ab, p1, b1, p2, b2):
    full = lambda a: pl.BlockSpec(a.shape, lambda i: (0,) * a.ndim)
    return pl.pallas_call(
        _readout_body,
        grid=(N // _RB,),
        in_specs=[pl.BlockSpec((_RB, H), lambda i: (i, 0)),
                  pl.BlockSpec((1, 1, _RB), lambda i: (i, 0, 0)),
                  full(aw), full(ab), full(p1), full(b1), full(p2), full(b2)],
        out_specs=pl.BlockSpec((G, 1), lambda i: (0, 0)),
        out_shape=jax.ShapeDtypeStruct((G, 1), jnp.float32),
        scratch_shapes=[pltpu.VMEM((G, H), jnp.float32),
                        pltpu.VMEM((G, H), jnp.float32)],
    )(nf, gid3, aw, ab, p1, b1, p2, b2)


def kernel(node_feats, edge_feats, params, edge_index, graph_ids):
    src3 = edge_index[0].reshape(_NW, _NCHUNK, _CH)
    dst3 = edge_index[1].reshape(_NW, _NCHUNK, _CH)
    dst3s = edge_index[1].reshape(_NW, _NCHUNK_S, _CHS)
    row = lambda v: v.reshape(1, -1)

    nf, ef = node_feats, edge_feats
    for p in params["layers"]:
        nn, cat = _node_proj(nf, p["w_n2n"], row(p["b_n2n"]),
                             p["w_l"], row(p["b_l"]),
                             p["w_r"], row(p["b_r"]))
        gs, gd = _gather_sc(cat, src3, dst3)
        e2n = _msg_mm(ef, p["w_e2n"], row(p["b_e2n"]))
        a0, a1 = _scatter_sc(e2n, dst3s)
        new_ef = _ue_mm(ef, gs, gd,
                        p["w_e2e"], row(p["b_e2e"]),
                        p["w_ue"], row(p["b_ue"]))
        nf = _node_update(nn, a0, a1, p["w_un"], row(p["b_un"]))
        ef = new_ef

    gid3 = graph_ids.reshape(N // _RB, 1, _RB)
    return _readout(nf, gid3, params["atom_w"], row(params["atom_b"]),
                    params["p1_w"], row(params["p1_b"]),
                    params["p2_w"], row(params["p2_b"]))
